# baseline (speedup 1.0000x reference)
import jax
import jax.numpy as jnp
from jax.experimental import pallas as pl
from jax.experimental.pallas import tpu as pltpu

N_Q = 8
K = 1024
D = 128
BT = 4096
H = 512
PAD_T = 24576


def _rvq_block(xT_ref, cb_ref, cbb_ref, cb3_ref, idx_ref, out_ref):
    chains = [xT_ref[:, c * H:(c + 1) * H] for c in range(BT // H)]
    inds = [[] for _ in chains]
    for q in range(N_Q):
        e = cb_ref[q]
        e_sq_half = 0.5 * jnp.sum(e * e, axis=1, keepdims=True)
        for c, r in enumerate(chains):
            xeT = jnp.dot(cbb_ref[q], r.astype(jnp.bfloat16),
                          preferred_element_type=jnp.float32)
            s = e_sq_half - xeT
            m = jnp.min(s, axis=0, keepdims=True)
            oh = (s == m).astype(jnp.bfloat16)
            p = jnp.dot(idx_ref[...], oh,
                        preferred_element_type=jnp.float32)
            ind = (p[0:1, :] * 256.0 + p[1:2, :]).astype(jnp.int32)
            inds[c].append(ind)
            if q < N_Q - 1:
                limbs = jnp.dot(cb3_ref[q], oh,
                                preferred_element_type=jnp.float32)
                quantT = (limbs[:D] + limbs[D:2 * D]) + limbs[2 * D:]
                chains[c] = r - quantT
    out_ref[...] = jnp.concatenate(
        [jnp.concatenate(ii, axis=0) for ii in inds], axis=1)


def kernel(x, codebooks):
    B, T, _ = x.shape
    n_tok = B * T
    xf = x.reshape(n_tok, D)
    xT = jnp.pad(xf, ((0, PAD_T - n_tok), (0, 0))).T
    cbb = codebooks.astype(jnp.bfloat16)
    mask = jnp.uint32(0xFFFF0000)
    bits = jax.lax.bitcast_convert_type(codebooks, jnp.uint32)
    hi_f = jax.lax.bitcast_convert_type(bits & mask, jnp.float32)
    mid_full = codebooks - hi_f
    mbits = jax.lax.bitcast_convert_type(mid_full, jnp.uint32)
    mid_f = jax.lax.bitcast_convert_type(mbits & mask, jnp.float32)
    lo_f = mid_full - mid_f
    hi = hi_f.astype(jnp.bfloat16)
    mid = mid_f.astype(jnp.bfloat16)
    lo = lo_f.astype(jnp.bfloat16)
    cb3 = jnp.concatenate([hi, mid, lo], axis=2)
    cb3 = cb3.transpose(0, 2, 1)
    ks = jnp.arange(K, dtype=jnp.int32)
    idx_tab = jnp.zeros((8, K), jnp.bfloat16)
    idx_tab = idx_tab.at[0].set((ks >> 8).astype(jnp.bfloat16))
    idx_tab = idx_tab.at[1].set((ks & 255).astype(jnp.bfloat16))
    grid = (PAD_T // BT,)
    codes = pl.pallas_call(
        _rvq_block,
        grid=grid,
        in_specs=[
            pl.BlockSpec((D, BT), lambda i: (0, i)),
            pl.BlockSpec((N_Q, K, D), lambda i: (0, 0, 0)),
            pl.BlockSpec((N_Q, K, D), lambda i: (0, 0, 0)),
            pl.BlockSpec((N_Q, 3 * D, K), lambda i: (0, 0, 0)),
            pl.BlockSpec((8, K), lambda i: (0, 0)),
        ],
        out_specs=pl.BlockSpec((N_Q, BT), lambda i: (0, i)),
        out_shape=jax.ShapeDtypeStruct((N_Q, PAD_T), jnp.int32),
        compiler_params=pltpu.CompilerParams(
            dimension_semantics=("arbitrary",)),
    )(xT, codebooks, cbb, cb3, idx_tab)
    return codes[:, :n_tok].reshape(N_Q, B, T).astype(jnp.int64)


# speedup vs baseline: 1.2013x; 1.2013x over previous
import jax
import jax.numpy as jnp
from jax.experimental import pallas as pl
from jax.experimental.pallas import tpu as pltpu

N_Q = 8
K = 1024
D = 128
BT = 2048
H = 512
PAD_T = 24576


def _rvq_block(x_ref, cb_ref, cbb_ref, cb3_ref, idx_ref, out_ref):
    chains = [x_ref[c * H:(c + 1) * H, :].T for c in range(BT // H)]
    inds = [[] for _ in chains]
    for q in range(N_Q):
        e = cb_ref[q]
        e_sq_half = 0.5 * jnp.sum(e * e, axis=1, keepdims=True)
        for c, r in enumerate(chains):
            xeT = jnp.dot(cbb_ref[q], r.astype(jnp.bfloat16),
                          preferred_element_type=jnp.float32)
            s = e_sq_half - xeT
            m = jnp.min(s, axis=0, keepdims=True)
            oh = (s == m).astype(jnp.bfloat16)
            p = jnp.dot(idx_ref[...], oh,
                        preferred_element_type=jnp.float32)
            ind = (p[0:1, :] * 256.0 + p[1:2, :]).astype(jnp.int32)
            inds[c].append(ind)
            if q < N_Q - 1:
                limbs = jnp.dot(cb3_ref[q], oh,
                                preferred_element_type=jnp.float32)
                quantT = (limbs[:D] + limbs[D:2 * D]) + limbs[2 * D:]
                chains[c] = r - quantT
    out_ref[...] = jnp.concatenate(
        [jnp.concatenate(ii, axis=0) for ii in inds], axis=1)


def kernel(x, codebooks):
    B, T, _ = x.shape
    n_tok = B * T
    xf = x.reshape(n_tok, D)
    cbb = codebooks.astype(jnp.bfloat16)
    mask = jnp.uint32(0xFFFF0000)
    bits = jax.lax.bitcast_convert_type(codebooks, jnp.uint32)
    hi_f = jax.lax.bitcast_convert_type(bits & mask, jnp.float32)
    mid_full = codebooks - hi_f
    mbits = jax.lax.bitcast_convert_type(mid_full, jnp.uint32)
    mid_f = jax.lax.bitcast_convert_type(mbits & mask, jnp.float32)
    lo_f = mid_full - mid_f
    hi = hi_f.astype(jnp.bfloat16)
    mid = mid_f.astype(jnp.bfloat16)
    lo = lo_f.astype(jnp.bfloat16)
    cb3 = jnp.concatenate([hi, mid, lo], axis=2)
    cb3 = cb3.transpose(0, 2, 1)
    ks = jnp.arange(K, dtype=jnp.int32)
    idx_tab = jnp.zeros((8, K), jnp.bfloat16)
    idx_tab = idx_tab.at[0].set((ks >> 8).astype(jnp.bfloat16))
    idx_tab = idx_tab.at[1].set((ks & 255).astype(jnp.bfloat16))
    grid = (PAD_T // BT,)
    codes = pl.pallas_call(
        _rvq_block,
        grid=grid,
        in_specs=[
            pl.BlockSpec((BT, D), lambda i: (i, 0)),
            pl.BlockSpec((N_Q, K, D), lambda i: (0, 0, 0)),
            pl.BlockSpec((N_Q, K, D), lambda i: (0, 0, 0)),
            pl.BlockSpec((N_Q, 3 * D, K), lambda i: (0, 0, 0)),
            pl.BlockSpec((8, K), lambda i: (0, 0)),
        ],
        out_specs=pl.BlockSpec((N_Q, BT), lambda i: (0, i)),
        out_shape=jax.ShapeDtypeStruct((N_Q, n_tok), jnp.int32),
        compiler_params=pltpu.CompilerParams(
            dimension_semantics=("arbitrary",)),
    )(xf, codebooks, cbb, cb3, idx_tab)
    return codes.reshape(N_Q, B, T).astype(jnp.int64)
